# Initial kernel scaffold; baseline (speedup 1.0000x reference)
import jax
import jax.numpy as jnp
from jax.experimental import pallas as pl


def kernel(wall_distances, keymask, key_embed):
    raise NotImplementedError("write your pallas kernel here")


# speedup vs baseline: 1.7831x; 1.7831x over previous
import functools

import jax
import jax.numpy as jnp
from jax import lax
from jax.experimental import pallas as pl
from jax.experimental.pallas import tpu as pltpu
from jax.experimental.pallas import tpu_sc as plsc

B = 1024 * 200
DW = 128
DE = 64
NW = 32
PER_W = B // NW
C = 640
NITER = PER_W // C
GSUB = C // 128

_mesh = plsc.VectorSubcoreMesh(core_axis_name="c", subcore_axis_name="s")


@functools.partial(
    pl.kernel,
    out_type=jax.ShapeDtypeStruct((B, DW + DE), jnp.float32),
    mesh=_mesh,
    scratch_types=[
        pltpu.VMEM((C,), jnp.int32),
        pltpu.VMEM((C, DW), jnp.float32),
        pltpu.VMEM((C, DE), jnp.float32),
        pltpu.SemaphoreType.DMA,
    ],
    compiler_params=pltpu.CompilerParams(use_tc_tiling_on_sc=False),
)
def _concat_gather(wall_hbm, idx_hbm, table_hbm, out_hbm, idx_v, wall_v, rows_v, sem):
    wid = lax.axis_index("s") * 2 + lax.axis_index("c")
    base = wid * PER_W

    def body(i, carry):
        off = base + i * C
        pltpu.sync_copy(idx_hbm.at[pl.ds(off, C)], idx_v)
        pltpu.sync_copy(wall_hbm.at[pl.ds(off, C), :], wall_v)
        for j in range(GSUB):
            pltpu.async_copy(
                table_hbm.at[idx_v.at[pl.ds(j * 128, 128)]],
                rows_v.at[pl.ds(j * 128, 128), :],
                sem,
            ).wait()
        pltpu.sync_copy(wall_v, out_hbm.at[pl.ds(off, C), pl.ds(0, DW)])
        pltpu.sync_copy(rows_v, out_hbm.at[pl.ds(off, C), pl.ds(DW, DE)])
        return carry

    lax.fori_loop(0, NITER, body, 0)


def kernel(wall_distances, keymask, key_embed):
    wall2d = wall_distances.reshape(B, DW)
    idx1d = keymask.reshape(B)
    out = _concat_gather(wall2d, idx1d, key_embed)
    return out.reshape(1024, 200, DW + DE)
